# Initial kernel scaffold; baseline (speedup 1.0000x reference)
import jax
import jax.numpy as jnp
from jax.experimental import pallas as pl


def kernel(fea, edge_index, edge_attr, edge_weights, l, W_hg, b_hg, W4, b4, W_ih, W_hh, b_ih, b_hh, Wq, bq, Wk, bk, Wv, bv, Wo, bo, Wc1, bc1, Wc2, bc2):
    raise NotImplementedError("write your pallas kernel here")


# speedup vs baseline: 6.0704x; 6.0704x over previous
import functools
import math

import jax
import jax.numpy as jnp
from jax import lax
from jax.experimental import pallas as pl
from jax.experimental.pallas import tpu as pltpu
from jax.experimental.pallas import tpu_sc as plsc

N = 4096
NE = 4096
E = 262144
D = 128
H = 4
DH = D // H

NC = 2
NS = 16
NW = NC * NS
LANES = 16

CH = 64
IDX_ROWS = E // CH
ROWS_PER_W = IDX_ROWS // NW
GROUPS = ROWS_PER_W // 2
SLICE = 4096 // NS


def _seg_compute(table, gi, si, out, gi_v, si_v, rows_v, z_v, sem, acc,
                 deg=None):
    c = lax.axis_index("c")
    s = lax.axis_index("s")
    wid = s * NC + c
    base = wid * ROWS_PER_W
    row0 = s * SLICE

    zeros16 = jnp.zeros((16,), jnp.float32)
    ones16 = jnp.ones((16,), jnp.float32)
    for r in range(CH):
        for k in range(D // 16):
            rows_v[0, r, pl.ds(k * 16, 16)] = zeros16
        if deg is not None:
            deg[0][r, :] = ones16
    for r in range(128):
        z_v[r, :] = zeros16

    for off in range(0, SLICE, CH):
        pltpu.sync_copy(rows_v.at[0], acc.at[pl.ds(row0 + off, CH)])
    if deg is not None:
        _, _, _, dg_acc, ds_acc = deg
        for off in range(0, SLICE, 128):
            pltpu.sync_copy(z_v, dg_acc.at[pl.ds(row0 + off, 128)])
            pltpu.sync_copy(z_v, ds_acc.at[pl.ds(row0 + off, 128)])
    plsc.subcore_barrier()

    def step(g, carry):
        r = base + g * 2
        pltpu.sync_copy(gi.at[pl.ds(r, 2)], gi_v)
        pltpu.sync_copy(si.at[pl.ds(r, 2)], si_v)
        d0 = pltpu.async_copy(table.at[gi_v.at[0]], rows_v.at[0], sem)
        d1 = pltpu.async_copy(table.at[gi_v.at[1]], rows_v.at[1], sem)
        for j, d in ((0, d0), (1, d1)):
            d.wait()
            pltpu.sync_copy(rows_v.at[j], acc.at[si_v.at[j]], add=True)
            if deg is not None:
                ones_v, _, _, dg_acc, ds_acc = deg
                pltpu.sync_copy(ones_v, dg_acc.at[gi_v.at[j]], add=True)
                pltpu.sync_copy(ones_v, ds_acc.at[si_v.at[j]], add=True)
        return carry

    lax.fori_loop(0, GROUPS, step, 0)
    plsc.subcore_barrier()

    pltpu.sync_copy(acc.at[pl.ds(row0, SLICE)], out.at[c, pl.ds(row0, SLICE)])
    if deg is not None:
        _, dg_out, ds_out, dg_acc, ds_acc = deg
        pltpu.sync_copy(dg_acc.at[pl.ds(row0, SLICE)],
                        dg_out.at[c, pl.ds(row0, SLICE)])
        pltpu.sync_copy(ds_acc.at[pl.ds(row0, SLICE)],
                        ds_out.at[c, pl.ds(row0, SLICE)])


def _seg_deg_body(table, gi, si, out, dg_out, ds_out,
                  gi_v, si_v, rows_v, z_v, ones_v, acc, dg_acc, ds_acc, sem):
    _seg_compute(table, gi, si, out, gi_v, si_v, rows_v, z_v, sem, acc,
                 deg=(ones_v, dg_out, ds_out, dg_acc, ds_acc))


def _seg_body(table, gi, si, out, gi_v, si_v, rows_v, z_v, acc, sem):
    _seg_compute(table, gi, si, out, gi_v, si_v, rows_v, z_v, sem, acc)


def _sc_mesh():
    return plsc.VectorSubcoreMesh(core_axis_name="c", subcore_axis_name="s",
                                  num_cores=NC, num_subcores=NS)


def _make_seg_call(with_deg):
    outs = [jax.ShapeDtypeStruct((NC, 4096, D), jnp.float32)]
    scratch = [
        pltpu.VMEM((2, CH), jnp.int32),
        pltpu.VMEM((2, CH), jnp.int32),
        pltpu.VMEM((2, CH, D), jnp.float32),
        pltpu.VMEM((128, 16), jnp.float32),
    ]
    if with_deg:
        outs += [jax.ShapeDtypeStruct((NC, 4096, 16), jnp.float32)] * 2
        scratch += [pltpu.VMEM((CH, 16), jnp.float32)]
    scratch += [pltpu.VMEM_SHARED((4096, D), jnp.float32)]
    if with_deg:
        scratch += [pltpu.VMEM_SHARED((4096, 16), jnp.float32)] * 2
    scratch += [pltpu.SemaphoreType.DMA]
    body = _seg_deg_body if with_deg else _seg_body
    return pl.kernel(body, out_type=tuple(outs), mesh=_sc_mesh(),
                     scratch_types=tuple(scratch))


def _lstm(z, wih, bih, bhh):
    gates = jnp.dot(z, wih.T, preferred_element_type=jnp.float32) + bih + bhh
    i = jax.nn.sigmoid(gates[:, :D])
    g = jnp.tanh(gates[:, 2 * D:3 * D])
    o = jax.nn.sigmoid(gates[:, 3 * D:])
    return o * jnp.tanh(i * g)


def _pre_body(i1, whg, w4, b4, wih, bih, bhh, xw_o, fea2_o):
    x1 = i1[...]
    xw_o[...] = jnp.dot(x1, whg[...].T, preferred_element_type=jnp.float32)
    z = jnp.dot(x1, w4[...].T, preferred_element_type=jnp.float32) + b4[...]
    fea2_o[...] = _lstm(z, wih[...], bih[...], bhh[...])


def _henorm_body(hp, dep, out):
    h = hp[...]
    de = jnp.sum(dep[...][0] + dep[...][1], axis=-1, keepdims=True) * (1.0 / 16.0)
    be = jnp.where(de > 0, 1.0 / de, 0.0)
    out[...] = (h[0] + h[1]) * be


QB = 512


def _att_body(xp, dvp, bhg, fea2, wih, bih, bhh, wq, bq, wk, bk, wv, bv,
              wo, bo, wc1, bc1, wc2, bc2, out, k_s, v_s):
    pi = pl.program_id(0)

    @pl.when(pi == 0)
    def _init():
        dv = jnp.sum(dvp[...][0] + dvp[...][1], axis=-1, keepdims=True) * (1.0 / 16.0)
        dvinv = jnp.where(dv > 0, 1.0 / dv, 0.0)
        xs = xp[...]
        x = (xs[0] + xs[1]) * dvinv + bhg[...]
        x = _lstm(x, wih[...], bih[...], bhh[...])
        k_s[...] = jnp.dot(x, wk[...].T, preferred_element_type=jnp.float32) + bk[...]
        v_s[...] = jnp.dot(fea2[...], wv[...].T,
                           preferred_element_type=jnp.float32) + bv[...]

    f2b = fea2[pl.ds(pi * QB, QB), :]
    Qb = jnp.dot(f2b, wq[...].T, preferred_element_type=jnp.float32) + bq[...]
    K = k_s[...]
    V = v_s[...]
    scale = 1.0 / math.sqrt(DH)
    ctxs = []
    for h in range(H):
        Qh = Qb[:, h * DH:(h + 1) * DH]
        Kh = K[:, h * DH:(h + 1) * DH]
        Vh = V[:, h * DH:(h + 1) * DH]
        S = jnp.dot(Qh, Kh.T, preferred_element_type=jnp.float32) * scale
        m = jnp.max(S, axis=-1, keepdims=True)
        P = jnp.exp(S - m)
        sm = jnp.sum(P, axis=-1, keepdims=True)
        ctxs.append(jnp.dot(P, Vh, preferred_element_type=jnp.float32) / sm)
    ctxb = jnp.concatenate(ctxs, axis=1)
    sh = jnp.dot(ctxb, wo[...].T, preferred_element_type=jnp.float32) + bo[...]
    h1 = jnp.maximum(
        jnp.dot(sh, wc1[...].T, preferred_element_type=jnp.float32) + bc1[...], 0.0)
    out[...] = jnp.dot(h1, wc2[...].T, preferred_element_type=jnp.float32) + bc2[...]


def _att_call(x_p, dv_p, b_hg, fea2, W_ih, b_ih, b_hh, Wq, bq, Wk, bk,
              Wv, bv, Wo, bo, Wc1, bc1, Wc2, bc2, interpret=False):
    full = lambda *shape: pl.BlockSpec(shape, lambda i: (0,) * len(shape))
    return pl.pallas_call(
        _att_body,
        grid=(N // QB,),
        in_specs=[
            full(NC, N, D),
            full(NC, N, 16),
            full(D),
            full(N, D),
            full(4 * D, D), full(4 * D), full(4 * D),
            full(D, D), full(D), full(D, D), full(D),
            full(D, D), full(D), full(D, D), full(D),
            full(D // 2, D), full(D // 2),
            full(2, D // 2), full(2),
        ],
        out_specs=pl.BlockSpec((QB, 2), lambda i: (i, 0)),
        out_shape=jax.ShapeDtypeStruct((N, 2), jnp.float32),
        scratch_shapes=[pltpu.VMEM((N, D), jnp.float32),
                        pltpu.VMEM((N, D), jnp.float32)],
        interpret=interpret,
    )(x_p, dv_p, b_hg, fea2, W_ih, b_ih, b_hh, Wq, bq, Wk, bk, Wv, bv,
      Wo, bo, Wc1, bc1, Wc2, bc2)


def kernel(fea, edge_index, edge_attr, edge_weights, l, W_hg, b_hg, W4, b4,
           W_ih, W_hh, b_ih, b_hh, Wq, bq, Wk, bk, Wv, bv, Wo, bo,
           Wc1, bc1, Wc2, bc2):
    input1 = fea[:, 1:]
    ni = edge_index[0].reshape(IDX_ROWS, CH)
    hi = edge_index[1].reshape(IDX_ROWS, CH)

    xw, fea2 = pl.pallas_call(
        _pre_body,
        out_shape=(jax.ShapeDtypeStruct((N, D), jnp.float32),
                   jax.ShapeDtypeStruct((N, D), jnp.float32)),
    )(input1, W_hg, W4, b4, W_ih, b_ih, b_hh)

    he_p, dv_p, de_p = _make_seg_call(True)(xw, ni, hi)

    he_norm = pl.pallas_call(
        _henorm_body,
        out_shape=jax.ShapeDtypeStruct((NE, D), jnp.float32),
    )(he_p, de_p)

    (x_p,) = _make_seg_call(False)(he_norm, hi, ni)

    out = _att_call(x_p, dv_p, b_hg, fea2, W_ih, b_ih, b_hh, Wq, bq, Wk, bk,
                    Wv, bv, Wo, bo, Wc1, bc1, Wc2, bc2)
    return out
